# Initial kernel scaffold; baseline (speedup 1.0000x reference)
import functools

import jax
import jax.numpy as jnp
from jax import lax
from jax.experimental import pallas as pl
from jax.experimental.pallas import tpu as pltpu
from jax.experimental.pallas import tpu_sc as plsc

NC = 2
NS = 16
NW = NC * NS
LANES = 16
K = 128
D = 128


def _agg_body(n_nodes, n_chunks, with_deg, *refs):
    if with_deg:
        (x_hbm, src_hbm, dst_hbm, ew_hbm, sums_out, degs_out,
         sum_sh, deg_sh, src_v, dst_v, ew_v, rows_v, ones_v) = refs
    else:
        (x_hbm, src_hbm, dst_hbm, ew_hbm, sums_out,
         sum_sh, src_v, dst_v, ew_v, rows_v) = refs
        deg_sh = ones_v = degs_out = None

    cid = lax.axis_index("c")
    sid = lax.axis_index("s")
    wid = sid * NC + cid

    rows_per_tile = n_nodes // NS
    zc = 125
    nz = rows_per_tile // zc

    def _zero_rows(r, c):
        for s in range(D // LANES):
            rows_v[r, pl.ds(s * LANES, LANES)] = jnp.zeros((LANES,), jnp.float32)
        return c
    lax.fori_loop(0, K, _zero_rows, 0)

    r0 = sid * rows_per_tile
    for j in range(nz):
        pltpu.sync_copy(rows_v.at[pl.ds(0, zc)],
                        sum_sh.at[pl.ds(r0 + j * zc, zc)])
    if with_deg:
        def _zero_ones(r, c):
            ones_v[r, pl.ds(0, LANES)] = jnp.zeros((LANES,), jnp.float32)
            return c
        lax.fori_loop(0, K, _zero_ones, 0)
        for j in range(nz):
            pltpu.sync_copy(ones_v.at[pl.ds(0, zc)],
                            deg_sh.at[pl.ds(r0 + j * zc, zc)])
        one_row = jnp.where(lax.iota(jnp.int32, LANES) == 0,
                            jnp.float32(1.0), jnp.float32(0.0))
        def _fill_ones(r, c):
            ones_v[r, pl.ds(0, LANES)] = one_row
            return c
        lax.fori_loop(0, K, _fill_ones, 0)

    plsc.subcore_barrier()

    n_c = n_chunks // NW + jnp.where(wid < n_chunks % NW, 1, 0)

    def _chunk(i, c):
        base = pl.multiple_of((wid + i * NW) * K, K)
        pltpu.sync_copy(src_hbm.at[pl.ds(base, K)], src_v)
        pltpu.sync_copy(dst_hbm.at[pl.ds(base, K)], dst_v)
        pltpu.sync_copy(ew_hbm.at[pl.ds(base, K)], ew_v)
        pltpu.sync_copy(x_hbm.at[src_v], rows_v)
        def _edge(e, cc):
            w = ew_v[e]
            for s in range(D // LANES):
                sl = pl.ds(s * LANES, LANES)
                rows_v[e, sl] = rows_v[e, sl] * w
            return cc
        lax.fori_loop(0, K, _edge, 0)
        pltpu.sync_copy(rows_v, sum_sh.at[dst_v], add=True)
        if with_deg:
            pltpu.sync_copy(ones_v, deg_sh.at[dst_v], add=True)
        return c
    lax.fori_loop(0, n_c, _chunk, 0)

    plsc.subcore_barrier()

    pltpu.sync_copy(sum_sh.at[pl.ds(r0, rows_per_tile)],
                    sums_out.at[cid, pl.ds(r0, rows_per_tile)])
    if with_deg:
        pltpu.sync_copy(deg_sh.at[pl.ds(r0, rows_per_tile)],
                        degs_out.at[cid, pl.ds(r0, rows_per_tile)])


def _make_agg(n_nodes, n_edges, with_deg):
    n_chunks = n_edges // K
    out_type = [jax.ShapeDtypeStruct((NC, n_nodes, D), jnp.float32)]
    scratch = [pltpu.VMEM_SHARED((n_nodes, D), jnp.float32)]
    if with_deg:
        out_type.append(jax.ShapeDtypeStruct((NC, n_nodes, LANES), jnp.float32))
        scratch.append(pltpu.VMEM_SHARED((n_nodes, LANES), jnp.float32))
    scratch += [
        pltpu.VMEM((K,), jnp.int32),
        pltpu.VMEM((K,), jnp.int32),
        pltpu.VMEM((K,), jnp.float32),
        pltpu.VMEM((K, D), jnp.float32),
    ]
    if with_deg:
        scratch.append(pltpu.VMEM((K, LANES), jnp.float32))
    mesh = plsc.VectorSubcoreMesh(core_axis_name="c", subcore_axis_name="s")
    return pl.kernel(
        functools.partial(_agg_body, n_nodes, n_chunks, with_deg),
        out_type=out_type,
        mesh=mesh,
        scratch_types=scratch,
    )


def _sage_tc_body(relu, x_ref, s_ref, d_ref, ws_ref, wn_ref, b_ref, o_ref):
    s = s_ref[0] + s_ref[1]
    deg = jnp.sum(d_ref[0] + d_ref[1], axis=1, keepdims=True)
    hn = s / jnp.maximum(deg, 1.0)
    acc = jnp.dot(x_ref[...], ws_ref[...], preferred_element_type=jnp.float32)
    acc = acc + jnp.dot(hn, wn_ref[...], preferred_element_type=jnp.float32)
    acc = acc + b_ref[...]
    o_ref[...] = jnp.maximum(acc, 0.0) if relu else acc


def _sage_tc(x, sums, degs, w_self, w_neigh, b, relu, rb=400):
    n, d = x.shape
    h = w_self.shape[1]
    grid = (n // rb,)
    return pl.pallas_call(
        functools.partial(_sage_tc_body, relu),
        grid=grid,
        in_specs=[
            pl.BlockSpec((rb, d), lambda i: (i, 0)),
            pl.BlockSpec((NC, rb, d), lambda i: (0, i, 0)),
            pl.BlockSpec((NC, rb, LANES), lambda i: (0, i, 0)),
            pl.BlockSpec((d, h), lambda i: (0, 0)),
            pl.BlockSpec((d, h), lambda i: (0, 0)),
            pl.BlockSpec((1, h), lambda i: (0, 0)),
        ],
        out_specs=pl.BlockSpec((rb, h), lambda i: (i, 0)),
        out_shape=jax.ShapeDtypeStruct((n, h), jnp.float32),
    )(x, sums, degs, w_self, w_neigh, b.reshape(1, -1))


def _head_body(x1_ref, s_ref, d_ref, w2s_ref, w2n_ref, b2_ref,
               l1w_ref, l1b_ref, l2w_ref, l2b_ref, l3w_ref, l3b_ref,
               out_ref, emb_ref):
    s = s_ref[0] + s_ref[1]
    deg = jnp.sum(d_ref[0] + d_ref[1], axis=1, keepdims=True)
    hn = s / jnp.maximum(deg, 1.0)
    emb = jnp.dot(x1_ref[...], w2s_ref[...], preferred_element_type=jnp.float32)
    emb = emb + jnp.dot(hn, w2n_ref[...], preferred_element_type=jnp.float32)
    emb = emb + b2_ref[...]
    emb_ref[...] = emb
    t = jnp.maximum(jnp.dot(emb, l1w_ref[...],
                            preferred_element_type=jnp.float32) + l1b_ref[...], 0.0)
    t = jnp.maximum(jnp.dot(t, l2w_ref[...],
                            preferred_element_type=jnp.float32) + l2b_ref[...], 0.0)
    out_ref[...] = jnp.dot(t, l3w_ref[...],
                           preferred_element_type=jnp.float32) + l3b_ref[...]


def _head_tc(x1, sums, degs, w2s, w2n, b2, l1w, l1b, l2w, l2b, l3w, l3b, rb=400):
    n, d = x1.shape
    h = w2s.shape[1]
    c = l3w.shape[1]
    h1 = l1w.shape[1]
    h2 = l2w.shape[1]
    grid = (n // rb,)

    def full(*shape):
        return pl.BlockSpec(shape, lambda i: tuple(0 for _ in shape))

    return pl.pallas_call(
        _head_body,
        grid=grid,
        in_specs=[
            pl.BlockSpec((rb, d), lambda i: (i, 0)),
            pl.BlockSpec((NC, rb, d), lambda i: (0, i, 0)),
            pl.BlockSpec((NC, rb, LANES), lambda i: (0, i, 0)),
            full(d, h), full(d, h), full(1, h),
            full(h, h1), full(1, h1),
            full(h1, h2), full(1, h2),
            full(h2, c), full(1, c),
        ],
        out_specs=[
            pl.BlockSpec((rb, c), lambda i: (i, 0)),
            pl.BlockSpec((rb, h), lambda i: (i, 0)),
        ],
        out_shape=[
            jax.ShapeDtypeStruct((n, c), jnp.float32),
            jax.ShapeDtypeStruct((n, h), jnp.float32),
        ],
    )(x1, sums, degs, w2s, w2n, b2.reshape(1, -1),
      l1w, l1b.reshape(1, -1), l2w, l2b.reshape(1, -1), l3w, l3b.reshape(1, -1))


def kernel(features, edge_index, edge_weights,
           W1_self, W1_neigh, b1, W2_self, W2_neigh, b2,
           L1w, L1b, L2w, L2b, L3w, L3b):
    n, d = features.shape
    e = edge_weights.shape[0]
    assert d == D and e % K == 0 and n % NS == 0
    src = edge_index[0]
    dst = edge_index[1]

    agg1 = _make_agg(n, e, with_deg=True)
    agg2 = _make_agg(n, e, with_deg=False)

    sums1, degs = agg1(features, src, dst, edge_weights)
    x1 = _sage_tc(features, sums1, degs, W1_self, W1_neigh, b1, relu=True)
    (sums2,) = agg2(x1, src, dst, edge_weights)
    out, emb = _head_tc(x1, sums2, degs, W2_self, W2_neigh, b2,
                        L1w, L1b, L2w, L2b, L3w, L3b)
    return (out, emb)


# speedup vs baseline: 4.9120x; 4.9120x over previous
import functools

import jax
import jax.numpy as jnp
from jax import lax
from jax.experimental import pallas as pl
from jax.experimental.pallas import tpu as pltpu
from jax.experimental.pallas import tpu_sc as plsc

NC = 2
NS = 16
NW = NC * NS
LANES = 16
K = 128
D = 128


def _agg_body(n_nodes, n_chunks, with_deg, *refs):
    if with_deg:
        (x_hbm, src_hbm, dst_hbm, ew_hbm, sums_out, degs_out,
         sum_sh, deg_sh, src_v, dst_v, ew_v, rows_v, ones_v, sem) = refs
    else:
        (x_hbm, src_hbm, dst_hbm, ew_hbm, sums_out,
         sum_sh, src_v, dst_v, ew_v, rows_v, sem) = refs
        deg_sh = ones_v = degs_out = None

    cid = lax.axis_index("c")
    sid = lax.axis_index("s")
    wid = sid * NC + cid

    ZU = 80
    CU = 400
    assert n_nodes % ZU == 0 and n_nodes % CU == 0 and ZU <= K
    nzu = n_nodes // ZU
    ncu = n_nodes // CU

    def _zero_rows(r, c):
        for s in range(D // LANES):
            rows_v[r, pl.ds(s * LANES, LANES)] = jnp.zeros((LANES,), jnp.float32)
        return c
    lax.fori_loop(0, K, _zero_rows, 0)
    if with_deg:
        def _zero_ones(r, c):
            ones_v[r, pl.ds(0, LANES)] = jnp.zeros((LANES,), jnp.float32)
            return c
        lax.fori_loop(0, K, _zero_ones, 0)

    n_zu = nzu // NS + jnp.where(sid < nzu % NS, 1, 0)

    def _zu(i, c):
        u = sid + i * NS
        pltpu.sync_copy(rows_v.at[pl.ds(0, ZU)], sum_sh.at[pl.ds(u * ZU, ZU)])
        if with_deg:
            pltpu.sync_copy(ones_v.at[pl.ds(0, ZU)],
                            deg_sh.at[pl.ds(u * ZU, ZU)])
        return c
    lax.fori_loop(0, n_zu, _zu, 0)

    if with_deg:
        one_row = jnp.where(lax.iota(jnp.int32, LANES) == 0,
                            jnp.float32(1.0), jnp.float32(0.0))
        def _fill_ones(r, c):
            ones_v[r, pl.ds(0, LANES)] = one_row
            return c
        lax.fori_loop(0, K, _fill_ones, 0)

    plsc.subcore_barrier()

    n_c = n_chunks // NW + jnp.where(wid < n_chunks % NW, 1, 0)

    def _chunk(i, c):
        base = pl.multiple_of((wid + i * NW) * K, K)
        pltpu.sync_copy(src_hbm.at[pl.ds(base, K)], src_v)
        pltpu.sync_copy(dst_hbm.at[pl.ds(base, K)], dst_v.at[0])
        pltpu.sync_copy(ew_hbm.at[pl.ds(base, K)], ew_v)
        pltpu.async_copy(x_hbm.at[src_v], rows_v, sem).wait()
        def _edge_group(g, cc):
            wv = ew_v[pl.ds(g * LANES, LANES)]
            for j in range(LANES):
                e = g * LANES + j
                w = wv[j]
                for s in range(D // LANES):
                    sl = pl.ds(s * LANES, LANES)
                    rows_v[e, sl] = rows_v[e, sl] * w
            return cc
        lax.fori_loop(0, K // LANES, _edge_group, 0)
        pltpu.sync_copy(rows_v, sum_sh.at[dst_v.at[0]], add=True)
        if with_deg:
            pltpu.sync_copy(ones_v, deg_sh.at[dst_v.at[0]], add=True)
        return c
    lax.fori_loop(0, n_c, _chunk, 0)

    plsc.subcore_barrier()

    n_cu = ncu // NS + jnp.where(sid < ncu % NS, 1, 0)

    def _cu(i, c):
        u = sid + i * NS
        pltpu.sync_copy(sum_sh.at[pl.ds(u * CU, CU)],
                        sums_out.at[cid, pl.ds(u * CU, CU)])
        if with_deg:
            pltpu.sync_copy(deg_sh.at[pl.ds(u * CU, CU)],
                            degs_out.at[cid, pl.ds(u * CU, CU)])
        return c
    lax.fori_loop(0, n_cu, _cu, 0)


def _make_agg(n_nodes, n_edges, with_deg):
    n_chunks = n_edges // K
    out_type = [jax.ShapeDtypeStruct((NC, n_nodes, D), jnp.float32)]
    scratch = [pltpu.VMEM_SHARED((n_nodes, D), jnp.float32)]
    if with_deg:
        out_type.append(jax.ShapeDtypeStruct((NC, n_nodes, LANES), jnp.float32))
        scratch.append(pltpu.VMEM_SHARED((n_nodes, LANES), jnp.float32))
    scratch += [
        pltpu.VMEM((K,), jnp.int32),
        pltpu.VMEM((1, K), jnp.int32),
        pltpu.VMEM((K,), jnp.float32),
        pltpu.VMEM((K, D), jnp.float32),
    ]
    if with_deg:
        scratch.append(pltpu.VMEM((K, LANES), jnp.float32))
    scratch.append(pltpu.SemaphoreType.DMA)
    mesh = plsc.VectorSubcoreMesh(core_axis_name="c", subcore_axis_name="s")
    return pl.kernel(
        functools.partial(_agg_body, n_nodes, n_chunks, with_deg),
        out_type=out_type,
        mesh=mesh,
        scratch_types=scratch,
        compiler_params=pltpu.CompilerParams(use_tc_tiling_on_sc=False),
    )


def _sage_tc_body(relu, x_ref, s_ref, d_ref, ws_ref, wn_ref, b_ref, o_ref):
    s = s_ref[0] + s_ref[1]
    deg = jnp.sum(d_ref[0] + d_ref[1], axis=1, keepdims=True)
    hn = s / jnp.maximum(deg, 1.0)
    acc = jnp.dot(x_ref[...], ws_ref[...], preferred_element_type=jnp.float32)
    acc = acc + jnp.dot(hn, wn_ref[...], preferred_element_type=jnp.float32)
    acc = acc + b_ref[...]
    o_ref[...] = jnp.maximum(acc, 0.0) if relu else acc


def _sage_tc(x, sums, degs, w_self, w_neigh, b, relu, rb=400):
    n, d = x.shape
    h = w_self.shape[1]
    grid = (n // rb,)
    return pl.pallas_call(
        functools.partial(_sage_tc_body, relu),
        grid=grid,
        in_specs=[
            pl.BlockSpec((rb, d), lambda i: (i, 0)),
            pl.BlockSpec((NC, rb, d), lambda i: (0, i, 0)),
            pl.BlockSpec((NC, rb, LANES), lambda i: (0, i, 0)),
            pl.BlockSpec((d, h), lambda i: (0, 0)),
            pl.BlockSpec((d, h), lambda i: (0, 0)),
            pl.BlockSpec((1, h), lambda i: (0, 0)),
        ],
        out_specs=pl.BlockSpec((rb, h), lambda i: (i, 0)),
        out_shape=jax.ShapeDtypeStruct((n, h), jnp.float32),
    )(x, sums, degs, w_self, w_neigh, b.reshape(1, -1))


def _head_body(x1_ref, s_ref, d_ref, w2s_ref, w2n_ref, b2_ref,
               l1w_ref, l1b_ref, l2w_ref, l2b_ref, l3w_ref, l3b_ref,
               out_ref, emb_ref):
    s = s_ref[0] + s_ref[1]
    deg = jnp.sum(d_ref[0] + d_ref[1], axis=1, keepdims=True)
    hn = s / jnp.maximum(deg, 1.0)
    emb = jnp.dot(x1_ref[...], w2s_ref[...], preferred_element_type=jnp.float32)
    emb = emb + jnp.dot(hn, w2n_ref[...], preferred_element_type=jnp.float32)
    emb = emb + b2_ref[...]
    emb_ref[...] = emb
    t = jnp.maximum(jnp.dot(emb, l1w_ref[...],
                            preferred_element_type=jnp.float32) + l1b_ref[...], 0.0)
    t = jnp.maximum(jnp.dot(t, l2w_ref[...],
                            preferred_element_type=jnp.float32) + l2b_ref[...], 0.0)
    out_ref[...] = jnp.dot(t, l3w_ref[...],
                           preferred_element_type=jnp.float32) + l3b_ref[...]


def _head_tc(x1, sums, degs, w2s, w2n, b2, l1w, l1b, l2w, l2b, l3w, l3b, rb=400):
    n, d = x1.shape
    h = w2s.shape[1]
    c = l3w.shape[1]
    h1 = l1w.shape[1]
    h2 = l2w.shape[1]
    grid = (n // rb,)

    def full(*shape):
        return pl.BlockSpec(shape, lambda i: tuple(0 for _ in shape))

    return pl.pallas_call(
        _head_body,
        grid=grid,
        in_specs=[
            pl.BlockSpec((rb, d), lambda i: (i, 0)),
            pl.BlockSpec((NC, rb, d), lambda i: (0, i, 0)),
            pl.BlockSpec((NC, rb, LANES), lambda i: (0, i, 0)),
            full(d, h), full(d, h), full(1, h),
            full(h, h1), full(1, h1),
            full(h1, h2), full(1, h2),
            full(h2, c), full(1, c),
        ],
        out_specs=[
            pl.BlockSpec((rb, c), lambda i: (i, 0)),
            pl.BlockSpec((rb, h), lambda i: (i, 0)),
        ],
        out_shape=[
            jax.ShapeDtypeStruct((n, c), jnp.float32),
            jax.ShapeDtypeStruct((n, h), jnp.float32),
        ],
    )(x1, sums, degs, w2s, w2n, b2.reshape(1, -1),
      l1w, l1b.reshape(1, -1), l2w, l2b.reshape(1, -1), l3w, l3b.reshape(1, -1))


def kernel(features, edge_index, edge_weights,
           W1_self, W1_neigh, b1, W2_self, W2_neigh, b2,
           L1w, L1b, L2w, L2b, L3w, L3b):
    n, d = features.shape
    e = edge_weights.shape[0]
    assert d == D and e % K == 0 and n % NS == 0
    src = edge_index[0]
    dst = edge_index[1]

    agg1 = _make_agg(n, e, with_deg=True)
    agg2 = _make_agg(n, e, with_deg=False)

    sums1, degs = agg1(features, src, dst, edge_weights)
    x1 = _sage_tc(features, sums1, degs, W1_self, W1_neigh, b1, relu=True)
    (sums2,) = agg2(x1, src, dst, edge_weights)
    out, emb = _head_tc(x1, sums2, degs, W2_self, W2_neigh, b2,
                        L1w, L1b, L2w, L2b, L3w, L3b)
    return (out, emb)


# speedup vs baseline: 6.0338x; 1.2284x over previous
import functools

import jax
import jax.numpy as jnp
from jax import lax
from jax.experimental import pallas as pl
from jax.experimental.pallas import tpu as pltpu
from jax.experimental.pallas import tpu_sc as plsc

NC = 2
NS = 16
NW = NC * NS
LANES = 16
K = 128
D = 128


def _agg_body(n_nodes, n_chunks, with_deg, *refs):
    if with_deg:
        (x_hbm, src_hbm, dst_hbm, ew_hbm, sums_out, degs_out,
         sum_sh, deg_sh, src_v, dst_v, ew_v, rows_v, ones_v, sem) = refs
    else:
        (x_hbm, src_hbm, dst_hbm, ew_hbm, sums_out,
         sum_sh, src_v, dst_v, ew_v, rows_v, sem) = refs
        deg_sh = ones_v = degs_out = None

    cid = lax.axis_index("c")
    sid = lax.axis_index("s")
    wid = sid * NC + cid

    ZU = 80
    CU = 400
    assert n_nodes % ZU == 0 and n_nodes % CU == 0 and ZU <= K
    nzu = n_nodes // ZU
    ncu = n_nodes // CU

    def _zero_rows(r, c):
        for s in range(D // LANES):
            rows_v[r, pl.ds(s * LANES, LANES)] = jnp.zeros((LANES,), jnp.float32)
        return c
    lax.fori_loop(0, K, _zero_rows, 0)
    if with_deg:
        def _zero_ones(r, c):
            ones_v[r, pl.ds(0, LANES)] = jnp.zeros((LANES,), jnp.float32)
            return c
        lax.fori_loop(0, K, _zero_ones, 0)

    n_zu = nzu // NS + jnp.where(sid < nzu % NS, 1, 0)

    def _zu(i, c):
        u = sid + i * NS
        pltpu.sync_copy(rows_v.at[pl.ds(0, ZU)], sum_sh.at[pl.ds(u * ZU, ZU)])
        if with_deg:
            pltpu.sync_copy(ones_v.at[pl.ds(0, ZU)],
                            deg_sh.at[pl.ds(u * ZU, ZU)])
        return c
    lax.fori_loop(0, n_zu, _zu, 0)

    if with_deg:
        one_row = jnp.where(lax.iota(jnp.int32, LANES) == 0,
                            jnp.float32(1.0), jnp.float32(0.0))
        def _fill_ones(r, c):
            ones_v[r, pl.ds(0, LANES)] = one_row
            return c
        lax.fori_loop(0, K, _fill_ones, 0)

    plsc.subcore_barrier()

    n_c = n_chunks // NW + jnp.where(wid < n_chunks % NW, 1, 0)

    def _chunk(i, c):
        base = pl.multiple_of((wid + i * NW) * K, K)
        d1 = pltpu.async_copy(src_hbm.at[pl.ds(base, K)], src_v, sem)
        d2 = pltpu.async_copy(dst_hbm.at[pl.ds(base, K)], dst_v.at[0], sem)
        d3 = pltpu.async_copy(ew_hbm.at[pl.ds(base, K)], ew_v, sem)
        d1.wait(); d2.wait(); d3.wait()
        pltpu.async_copy(x_hbm.at[src_v], rows_v, sem).wait()
        def _edge_group(g, cc):
            wv = ew_v[pl.ds(g * LANES, LANES)]
            for j in range(LANES):
                e = g * LANES + j
                w = wv[j]
                for s in range(D // LANES):
                    sl = pl.ds(s * LANES, LANES)
                    rows_v[e, sl] = rows_v[e, sl] * w
            return cc
        lax.fori_loop(0, K // LANES, _edge_group, 0)
        s1 = pltpu.async_copy(rows_v, sum_sh.at[dst_v.at[0]], sem, add=True)
        if with_deg:
            s2 = pltpu.async_copy(ones_v, deg_sh.at[dst_v.at[0]], sem, add=True)
        s1.wait()
        if with_deg:
            s2.wait()
        return c
    lax.fori_loop(0, n_c, _chunk, 0)

    plsc.subcore_barrier()

    n_cu = ncu // NS + jnp.where(sid < ncu % NS, 1, 0)

    def _cu(i, c):
        u = sid + i * NS
        pltpu.sync_copy(sum_sh.at[pl.ds(u * CU, CU)],
                        sums_out.at[cid, pl.ds(u * CU, CU)])
        if with_deg:
            pltpu.sync_copy(deg_sh.at[pl.ds(u * CU, CU)],
                            degs_out.at[cid, pl.ds(u * CU, CU)])
        return c
    lax.fori_loop(0, n_cu, _cu, 0)


def _make_agg(n_nodes, n_edges, with_deg):
    n_chunks = n_edges // K
    out_type = [jax.ShapeDtypeStruct((NC, n_nodes, D), jnp.float32)]
    scratch = [pltpu.VMEM_SHARED((n_nodes, D), jnp.float32)]
    if with_deg:
        out_type.append(jax.ShapeDtypeStruct((NC, n_nodes, LANES), jnp.float32))
        scratch.append(pltpu.VMEM_SHARED((n_nodes, LANES), jnp.float32))
    scratch += [
        pltpu.VMEM((K,), jnp.int32),
        pltpu.VMEM((1, K), jnp.int32),
        pltpu.VMEM((K,), jnp.float32),
        pltpu.VMEM((K, D), jnp.float32),
    ]
    if with_deg:
        scratch.append(pltpu.VMEM((K, LANES), jnp.float32))
    scratch.append(pltpu.SemaphoreType.DMA)
    mesh = plsc.VectorSubcoreMesh(core_axis_name="c", subcore_axis_name="s")
    return pl.kernel(
        functools.partial(_agg_body, n_nodes, n_chunks, with_deg),
        out_type=out_type,
        mesh=mesh,
        scratch_types=scratch,
        compiler_params=pltpu.CompilerParams(use_tc_tiling_on_sc=False),
    )


def _sage_tc_body(relu, x_ref, s_ref, d_ref, ws_ref, wn_ref, b_ref, o_ref):
    s = s_ref[0] + s_ref[1]
    deg = jnp.sum(d_ref[0] + d_ref[1], axis=1, keepdims=True)
    hn = s / jnp.maximum(deg, 1.0)
    acc = jnp.dot(x_ref[...], ws_ref[...], preferred_element_type=jnp.float32)
    acc = acc + jnp.dot(hn, wn_ref[...], preferred_element_type=jnp.float32)
    acc = acc + b_ref[...]
    o_ref[...] = jnp.maximum(acc, 0.0) if relu else acc


def _sage_tc(x, sums, degs, w_self, w_neigh, b, relu, rb=400):
    n, d = x.shape
    h = w_self.shape[1]
    grid = (n // rb,)
    return pl.pallas_call(
        functools.partial(_sage_tc_body, relu),
        grid=grid,
        in_specs=[
            pl.BlockSpec((rb, d), lambda i: (i, 0)),
            pl.BlockSpec((NC, rb, d), lambda i: (0, i, 0)),
            pl.BlockSpec((NC, rb, LANES), lambda i: (0, i, 0)),
            pl.BlockSpec((d, h), lambda i: (0, 0)),
            pl.BlockSpec((d, h), lambda i: (0, 0)),
            pl.BlockSpec((1, h), lambda i: (0, 0)),
        ],
        out_specs=pl.BlockSpec((rb, h), lambda i: (i, 0)),
        out_shape=jax.ShapeDtypeStruct((n, h), jnp.float32),
    )(x, sums, degs, w_self, w_neigh, b.reshape(1, -1))


def _head_body(x1_ref, s_ref, d_ref, w2s_ref, w2n_ref, b2_ref,
               l1w_ref, l1b_ref, l2w_ref, l2b_ref, l3w_ref, l3b_ref,
               out_ref, emb_ref):
    s = s_ref[0] + s_ref[1]
    deg = jnp.sum(d_ref[0] + d_ref[1], axis=1, keepdims=True)
    hn = s / jnp.maximum(deg, 1.0)
    emb = jnp.dot(x1_ref[...], w2s_ref[...], preferred_element_type=jnp.float32)
    emb = emb + jnp.dot(hn, w2n_ref[...], preferred_element_type=jnp.float32)
    emb = emb + b2_ref[...]
    emb_ref[...] = emb
    t = jnp.maximum(jnp.dot(emb, l1w_ref[...],
                            preferred_element_type=jnp.float32) + l1b_ref[...], 0.0)
    t = jnp.maximum(jnp.dot(t, l2w_ref[...],
                            preferred_element_type=jnp.float32) + l2b_ref[...], 0.0)
    out_ref[...] = jnp.dot(t, l3w_ref[...],
                           preferred_element_type=jnp.float32) + l3b_ref[...]


def _head_tc(x1, sums, degs, w2s, w2n, b2, l1w, l1b, l2w, l2b, l3w, l3b, rb=400):
    n, d = x1.shape
    h = w2s.shape[1]
    c = l3w.shape[1]
    h1 = l1w.shape[1]
    h2 = l2w.shape[1]
    grid = (n // rb,)

    def full(*shape):
        return pl.BlockSpec(shape, lambda i: tuple(0 for _ in shape))

    return pl.pallas_call(
        _head_body,
        grid=grid,
        in_specs=[
            pl.BlockSpec((rb, d), lambda i: (i, 0)),
            pl.BlockSpec((NC, rb, d), lambda i: (0, i, 0)),
            pl.BlockSpec((NC, rb, LANES), lambda i: (0, i, 0)),
            full(d, h), full(d, h), full(1, h),
            full(h, h1), full(1, h1),
            full(h1, h2), full(1, h2),
            full(h2, c), full(1, c),
        ],
        out_specs=[
            pl.BlockSpec((rb, c), lambda i: (i, 0)),
            pl.BlockSpec((rb, h), lambda i: (i, 0)),
        ],
        out_shape=[
            jax.ShapeDtypeStruct((n, c), jnp.float32),
            jax.ShapeDtypeStruct((n, h), jnp.float32),
        ],
    )(x1, sums, degs, w2s, w2n, b2.reshape(1, -1),
      l1w, l1b.reshape(1, -1), l2w, l2b.reshape(1, -1), l3w, l3b.reshape(1, -1))


def kernel(features, edge_index, edge_weights,
           W1_self, W1_neigh, b1, W2_self, W2_neigh, b2,
           L1w, L1b, L2w, L2b, L3w, L3b):
    n, d = features.shape
    e = edge_weights.shape[0]
    assert d == D and e % K == 0 and n % NS == 0
    src = edge_index[0]
    dst = edge_index[1]

    agg1 = _make_agg(n, e, with_deg=True)
    agg2 = _make_agg(n, e, with_deg=False)

    sums1, degs = agg1(features, src, dst, edge_weights)
    x1 = _sage_tc(features, sums1, degs, W1_self, W1_neigh, b1, relu=True)
    (sums2,) = agg2(x1, src, dst, edge_weights)
    out, emb = _head_tc(x1, sums2, degs, W2_self, W2_neigh, b2,
                        L1w, L1b, L2w, L2b, L3w, L3b)
    return (out, emb)
